# Initial kernel scaffold; baseline (speedup 1.0000x reference)
import jax
import jax.numpy as jnp
from jax.experimental import pallas as pl


def kernel(encoder_output, decoder_output, lengths, fc1_w, fc1_b, fc2_w, fc2_b, T, batch_size, output_weights):
    raise NotImplementedError("write your pallas kernel here")


# speedup vs baseline: 3.0799x; 3.0799x over previous
import functools

import jax
import jax.numpy as jnp
from jax.experimental import pallas as pl
from jax.experimental.pallas import tpu as pltpu

WINDOW_SIZE = 128
WL = 2 * WINDOW_SIZE + 1
TWO_STD_SQ = 2.0 * (WINDOW_SIZE / 2.0) ** 2

_DOT = functools.partial(
    jax.lax.dot_general,
    preferred_element_type=jnp.float32,
)


def _k0_body(dec_ref, w1_ref, b1_ref, w2_ref, b2_ref, len_ref, st_ref):
    x = dec_ref[...]
    g = _DOT(x, w1_ref[...], (((1,), (1,)), ((), ())))
    t1 = jnp.tanh(g + b1_ref[...])
    z = _DOT(t1, w2_ref[...], (((1,), (1,)), ((), ())))[:, 0:1]
    sig = jax.nn.sigmoid(z + b2_ref[0])
    st_ref[...] = jnp.round(len_ref[...] * sig).astype(jnp.int32)


def _att_body(start_ref, len_ref, enc_ref, dec_ref, w1_ref, b1_ref, w2_ref,
              b2_ref, out_ref, ht_v, sel_v, out_v, sem_ht, sem_sel, sem_out):
    b = pl.program_id(0)

    ht_copy = pltpu.make_async_copy(dec_ref.at[:, b, :], ht_v, sem_ht)
    ht_copy.start()
    st = start_ref[b]
    sel_copy = pltpu.make_async_copy(
        enc_ref.at[pl.ds(st, WL), b, :], sel_v, sem_sel)
    sel_copy.start()

    ht_copy.wait()
    ht = ht_v[...]
    g = _DOT(ht, w1_ref[...], (((1,), (1,)), ((), ())))
    t1 = jnp.tanh(g + b1_ref[...])
    z = _DOT(t1, w2_ref[...], (((1,), (1,)), ((), ())))[:, 0:1]
    sig = jax.nn.sigmoid(z + b2_ref[0])
    length = len_ref[b].astype(jnp.float32)
    p = WINDOW_SIZE + length * sig
    ws = jnp.round(p - WINDOW_SIZE).astype(jnp.int32)

    T = ht.shape[0]
    iw = jax.lax.broadcasted_iota(jnp.int32, (T, WL), 1)
    pos = ws.astype(jnp.float32) + iw.astype(jnp.float32)
    gauss = jnp.exp(-((pos - p) ** 2) / TWO_STD_SQ)

    sel_copy.wait()
    sel = sel_v[...]
    score = _DOT(ht, sel, (((1,), (1,)), ((), ())))
    left = iw < (WINDOW_SIZE - ws)
    right = iw >= (len_ref[b] + WINDOW_SIZE - ws)
    score = jnp.where(left | right, jnp.float32(1e-14), score)
    m = jnp.max(score, axis=1, keepdims=True)
    e = jnp.exp(score - m)
    a = (e / jnp.sum(e, axis=1, keepdims=True)) * gauss
    out_v[...] = _DOT(a, sel, (((1,), (0,)), ((), ())))

    out_copy = pltpu.make_async_copy(out_v, out_ref.at[:, b, :], sem_out)
    out_copy.start()
    out_copy.wait()


def kernel(encoder_output, decoder_output, lengths, fc1_w, fc1_b, fc2_w,
           fc2_b, T, batch_size, output_weights):
    S, B, H = encoder_output.shape
    Tn = decoder_output.shape[0]
    H2 = fc1_w.shape[0]

    b1 = fc1_b.reshape(1, H2)
    w2 = jnp.zeros((8, H2), jnp.float32).at[0].set(fc2_w.reshape(H2))
    b2 = fc2_b.reshape(1)
    len_f = lengths.astype(jnp.float32).reshape(B, 1)

    start_last = pl.pallas_call(
        _k0_body,
        out_shape=jax.ShapeDtypeStruct((B, 1), jnp.int32),
        in_specs=[
            pl.BlockSpec(memory_space=pltpu.VMEM),
            pl.BlockSpec(memory_space=pltpu.VMEM),
            pl.BlockSpec(memory_space=pltpu.VMEM),
            pl.BlockSpec(memory_space=pltpu.VMEM),
            pl.BlockSpec(memory_space=pltpu.SMEM),
            pl.BlockSpec(memory_space=pltpu.VMEM),
        ],
        out_specs=pl.BlockSpec(memory_space=pltpu.VMEM),
    )(decoder_output[Tn - 1], fc1_w, b1, w2, b2, len_f)

    out = pl.pallas_call(
        _att_body,
        grid=(B,),
        in_specs=[
            pl.BlockSpec(memory_space=pltpu.SMEM),
            pl.BlockSpec(memory_space=pltpu.SMEM),
            pl.BlockSpec(memory_space=pl.ANY),
            pl.BlockSpec(memory_space=pl.ANY),
            pl.BlockSpec(memory_space=pltpu.VMEM),
            pl.BlockSpec(memory_space=pltpu.VMEM),
            pl.BlockSpec(memory_space=pltpu.VMEM),
            pl.BlockSpec(memory_space=pltpu.SMEM),
        ],
        out_specs=pl.BlockSpec(memory_space=pl.ANY),
        out_shape=jax.ShapeDtypeStruct((Tn, B, H), jnp.float32),
        scratch_shapes=[
            pltpu.VMEM((Tn, H), jnp.float32),
            pltpu.VMEM((WL, H), jnp.float32),
            pltpu.VMEM((Tn, H), jnp.float32),
            pltpu.SemaphoreType.DMA,
            pltpu.SemaphoreType.DMA,
            pltpu.SemaphoreType.DMA,
        ],
    )(start_last.reshape(B), lengths, encoder_output, decoder_output,
      fc1_w, b1, w2, b2)
    return out


# speedup vs baseline: 4.6299x; 1.5033x over previous
import functools

import jax
import jax.numpy as jnp
from jax.experimental import pallas as pl
from jax.experimental.pallas import tpu as pltpu

WINDOW_SIZE = 128
WL = 2 * WINDOW_SIZE + 1
TWO_STD_SQ = 2.0 * (WINDOW_SIZE / 2.0) ** 2

_DOT = functools.partial(
    jax.lax.dot_general,
    preferred_element_type=jnp.float32,
)


def _k0_body(dec_ref, w1_ref, b1_ref, w2_ref, b2_ref, len_ref, st_ref):
    x = dec_ref[...]
    g = _DOT(x, w1_ref[...], (((1,), (1,)), ((), ())))
    t1 = jnp.tanh(g + b1_ref[...])
    z = _DOT(t1, w2_ref[...], (((1,), (1,)), ((), ())))[:, 0:1]
    sig = jax.nn.sigmoid(z + b2_ref[0])
    st_ref[...] = jnp.round(len_ref[...] * sig).astype(jnp.int32)


def _att_body(start_ref, len_ref, enc_ref, dec_ref, w1_ref, b1_ref, w2_ref,
              b2_ref, out_ref, ht_v, sel_v, out_v, sem_ht, sem_sel, sem_out):
    b = pl.program_id(0)
    nb = pl.num_programs(0)
    B = nb

    def ht_copy(bb, buf):
        return pltpu.make_async_copy(
            dec_ref.at[:, bb, :], ht_v.at[buf], sem_ht.at[buf])

    def sel_copy(bb):
        return pltpu.make_async_copy(
            enc_ref.at[pl.ds(start_ref[bb], WL), bb, :], sel_v.at[bb], sem_sel)

    def out_copy(bb):
        return pltpu.make_async_copy(
            out_v.at[bb % 2], out_ref.at[:, bb, :], sem_out.at[bb % 2])

    @pl.when(b == 0)
    def _():
        ht_copy(0, 0).start()
        for bb in range(B):
            sel_copy(bb).start()

    @pl.when(b + 1 < nb)
    def _():
        ht_copy(b + 1, (b + 1) % 2).start()

    ht_copy(b, b % 2).wait()
    ht = ht_v[b % 2]
    g = _DOT(ht, w1_ref[...], (((1,), (1,)), ((), ())))
    t1 = jnp.tanh(g + b1_ref[...])
    z = _DOT(t1, w2_ref[...], (((1,), (1,)), ((), ())))[:, 0:1]
    sig = jax.nn.sigmoid(z + b2_ref[0])
    length = len_ref[b].astype(jnp.float32)
    p = WINDOW_SIZE + length * sig
    ws = jnp.round(p - WINDOW_SIZE).astype(jnp.int32)

    T = ht.shape[0]
    iw = jax.lax.broadcasted_iota(jnp.int32, (T, WL), 1)
    pos = ws.astype(jnp.float32) + iw.astype(jnp.float32)
    gauss = jnp.exp(-((pos - p) ** 2) / TWO_STD_SQ)

    @pl.when(b == 0)
    def _():
        for bb in range(B):
            sel_copy(bb).wait()

    sel = sel_v[b]
    score = _DOT(ht, sel, (((1,), (1,)), ((), ())))
    left = iw < (WINDOW_SIZE - ws)
    right = iw >= (len_ref[b] + WINDOW_SIZE - ws)
    score = jnp.where(left | right, jnp.float32(1e-14), score)
    m = jnp.max(score, axis=1, keepdims=True)
    e = jnp.exp(score - m)
    a = (e / jnp.sum(e, axis=1, keepdims=True)) * gauss

    @pl.when(b >= 2)
    def _():
        out_copy(b - 2).wait()

    out_v[b % 2] = _DOT(a, sel, (((1,), (0,)), ((), ())))
    out_copy(b).start()

    @pl.when(b == nb - 1)
    def _():
        out_copy(b - 1).wait()
        out_copy(b).wait()


def kernel(encoder_output, decoder_output, lengths, fc1_w, fc1_b, fc2_w,
           fc2_b, T, batch_size, output_weights):
    S, B, H = encoder_output.shape
    Tn = decoder_output.shape[0]
    H2 = fc1_w.shape[0]

    b1 = fc1_b.reshape(1, H2)
    w2 = jnp.zeros((8, H2), jnp.float32).at[0].set(fc2_w.reshape(H2))
    b2 = fc2_b.reshape(1)
    len_f = lengths.astype(jnp.float32).reshape(B, 1)

    start_last = pl.pallas_call(
        _k0_body,
        out_shape=jax.ShapeDtypeStruct((B, 1), jnp.int32),
        in_specs=[
            pl.BlockSpec(memory_space=pltpu.VMEM),
            pl.BlockSpec(memory_space=pltpu.VMEM),
            pl.BlockSpec(memory_space=pltpu.VMEM),
            pl.BlockSpec(memory_space=pltpu.VMEM),
            pl.BlockSpec(memory_space=pltpu.SMEM),
            pl.BlockSpec(memory_space=pltpu.VMEM),
        ],
        out_specs=pl.BlockSpec(memory_space=pltpu.VMEM),
    )(decoder_output[Tn - 1], fc1_w, b1, w2, b2, len_f)

    out = pl.pallas_call(
        _att_body,
        grid=(B,),
        in_specs=[
            pl.BlockSpec(memory_space=pltpu.SMEM),
            pl.BlockSpec(memory_space=pltpu.SMEM),
            pl.BlockSpec(memory_space=pl.ANY),
            pl.BlockSpec(memory_space=pl.ANY),
            pl.BlockSpec(memory_space=pltpu.VMEM),
            pl.BlockSpec(memory_space=pltpu.VMEM),
            pl.BlockSpec(memory_space=pltpu.VMEM),
            pl.BlockSpec(memory_space=pltpu.SMEM),
        ],
        out_specs=pl.BlockSpec(memory_space=pl.ANY),
        out_shape=jax.ShapeDtypeStruct((Tn, B, H), jnp.float32),
        scratch_shapes=[
            pltpu.VMEM((2, Tn, H), jnp.float32),
            pltpu.VMEM((B, WL, H), jnp.float32),
            pltpu.VMEM((2, Tn, H), jnp.float32),
            pltpu.SemaphoreType.DMA((2,)),
            pltpu.SemaphoreType.DMA,
            pltpu.SemaphoreType.DMA((2,)),
        ],
    )(start_last.reshape(B), lengths, encoder_output, decoder_output,
      fc1_w, b1, w2, b2)
    return out


# speedup vs baseline: 5.0726x; 1.0956x over previous
import functools

import jax
import jax.numpy as jnp
from jax.experimental import pallas as pl
from jax.experimental.pallas import tpu as pltpu

WINDOW_SIZE = 128
WL = 2 * WINDOW_SIZE + 1
TWO_STD_SQ = 2.0 * (WINDOW_SIZE / 2.0) ** 2

_DOT = functools.partial(
    jax.lax.dot_general,
    preferred_element_type=jnp.float32,
)


def _k0_body(dec_ref, w1_ref, b1_ref, w2_ref, b2_ref, len_ref, st_ref):
    x = dec_ref[...]
    g = _DOT(x, w1_ref[...], (((1,), (1,)), ((), ())))
    t1 = jnp.tanh(g + b1_ref[...])
    z = _DOT(t1, w2_ref[...], (((1,), (1,)), ((), ())))[:, 0:1]
    sig = jax.nn.sigmoid(z + b2_ref[0])
    st_ref[...] = jnp.round(len_ref[...] * sig).astype(jnp.int32)


def _att_body(start_ref, len_ref, enc_ref, dec_ref, w1_ref, b1_ref, w2_ref,
              b2_ref, out_ref, ht_v, sel_v, out_v, sem_ht, sem_sel, sem_out):
    b = pl.program_id(0)
    nb = pl.num_programs(0)
    B = nb

    def ht_copy(bb):
        return pltpu.make_async_copy(
            dec_ref.at[:, bb, :], ht_v.at[bb], sem_ht.at[bb])

    def sel_copy(bb):
        return pltpu.make_async_copy(
            enc_ref.at[pl.ds(start_ref[bb], WL), bb, :], sel_v.at[bb],
            sem_sel.at[bb])

    def out_copy(bb):
        return pltpu.make_async_copy(
            out_v.at[bb % 2], out_ref.at[:, bb, :], sem_out.at[bb % 2])

    @pl.when(b == 0)
    def _():
        ht_copy(0).start()
        sel_copy(0).start()
        ht_copy(1).start()
        for bb in range(1, B):
            sel_copy(bb).start()
        for bb in range(2, B):
            ht_copy(bb).start()

    ht_copy(b).wait()
    ht = ht_v[b]
    g = _DOT(ht, w1_ref[...], (((1,), (1,)), ((), ())))
    t1 = jnp.tanh(g + b1_ref[...])
    z = _DOT(t1, w2_ref[...], (((1,), (1,)), ((), ())))[:, 0:1]
    sig = jax.nn.sigmoid(z + b2_ref[0])
    length = len_ref[b].astype(jnp.float32)
    p = WINDOW_SIZE + length * sig
    ws = jnp.round(p - WINDOW_SIZE).astype(jnp.int32)

    T = ht.shape[0]
    iw = jax.lax.broadcasted_iota(jnp.int32, (T, WL), 1)
    pos = ws.astype(jnp.float32) + iw.astype(jnp.float32)
    gauss = jnp.exp(-((pos - p) ** 2) / TWO_STD_SQ)

    sel_copy(b).wait()
    sel = sel_v[b]
    score = _DOT(ht, sel, (((1,), (1,)), ((), ())))
    left = iw < (WINDOW_SIZE - ws)
    right = iw >= (len_ref[b] + WINDOW_SIZE - ws)
    score = jnp.where(left | right, jnp.float32(1e-14), score)
    m = jnp.max(score, axis=1, keepdims=True)
    e = jnp.exp(score - m)
    a = (e / jnp.sum(e, axis=1, keepdims=True)) * gauss

    @pl.when(b >= 2)
    def _():
        out_copy(b - 2).wait()

    c = _DOT(a[:, :WL - 1], sel[:WL - 1],
             (((1,), (0,)), ((), ())))
    c = c + a[:, WL - 1:WL] * sel[WL - 1:WL, :]
    out_v[b % 2] = c
    out_copy(b).start()

    @pl.when(b == nb - 1)
    def _():
        out_copy(b - 1).wait()
        out_copy(b).wait()


def kernel(encoder_output, decoder_output, lengths, fc1_w, fc1_b, fc2_w,
           fc2_b, T, batch_size, output_weights):
    S, B, H = encoder_output.shape
    Tn = decoder_output.shape[0]
    H2 = fc1_w.shape[0]

    b1 = fc1_b.reshape(1, H2)
    w2 = jnp.zeros((8, H2), jnp.float32).at[0].set(fc2_w.reshape(H2))
    b2 = fc2_b.reshape(1)
    len_f = lengths.astype(jnp.float32).reshape(B, 1)

    start_last = pl.pallas_call(
        _k0_body,
        out_shape=jax.ShapeDtypeStruct((B, 1), jnp.int32),
        in_specs=[
            pl.BlockSpec(memory_space=pltpu.VMEM),
            pl.BlockSpec(memory_space=pltpu.VMEM),
            pl.BlockSpec(memory_space=pltpu.VMEM),
            pl.BlockSpec(memory_space=pltpu.VMEM),
            pl.BlockSpec(memory_space=pltpu.SMEM),
            pl.BlockSpec(memory_space=pltpu.VMEM),
        ],
        out_specs=pl.BlockSpec(memory_space=pltpu.VMEM),
    )(decoder_output[Tn - 1], fc1_w, b1, w2, b2, len_f)

    out = pl.pallas_call(
        _att_body,
        grid=(B,),
        in_specs=[
            pl.BlockSpec(memory_space=pltpu.SMEM),
            pl.BlockSpec(memory_space=pltpu.SMEM),
            pl.BlockSpec(memory_space=pl.ANY),
            pl.BlockSpec(memory_space=pl.ANY),
            pl.BlockSpec(memory_space=pltpu.VMEM),
            pl.BlockSpec(memory_space=pltpu.VMEM),
            pl.BlockSpec(memory_space=pltpu.VMEM),
            pl.BlockSpec(memory_space=pltpu.SMEM),
        ],
        out_specs=pl.BlockSpec(memory_space=pl.ANY),
        out_shape=jax.ShapeDtypeStruct((Tn, B, H), jnp.float32),
        scratch_shapes=[
            pltpu.VMEM((B, Tn, H), jnp.float32),
            pltpu.VMEM((B, WL, H), jnp.float32),
            pltpu.VMEM((2, Tn, H), jnp.float32),
            pltpu.SemaphoreType.DMA((B,)),
            pltpu.SemaphoreType.DMA((B,)),
            pltpu.SemaphoreType.DMA((2,)),
        ],
    )(start_last.reshape(B), lengths, encoder_output, decoder_output,
      fc1_w, b1, w2, b2)
    return out
